# baseline (speedup 1.0000x reference)
import functools

import jax
import jax.numpy as jnp
from jax import lax
from jax.experimental import pallas as pl
from jax.experimental.pallas import tpu as pltpu
from jax.experimental.pallas import tpu_sc as plsc

EMB_DIM = 16
LANES = 16
NUM_CORES = 2
NUM_SUBCORES = 16
NUM_WORKERS = NUM_CORES * NUM_SUBCORES
CHUNK = 1024
COMB_ROWS = 96 * 8


def _splat(v):
    return jnp.full((LANES,), v, jnp.int32)


@functools.lru_cache(maxsize=None)
def _build_sc_call(b_dim, l_dim):
    n = b_dim * l_dim
    assert b_dim % CHUNK == 0 and EMB_DIM == 16 and b_dim % 128 == 0
    s_per_l = b_dim // CHUNK
    total_chunks = l_dim * s_per_l
    assert total_chunks % NUM_WORKERS == 0
    n_chunks = total_chunks // NUM_WORKERS
    l_stride = b_dim * EMB_DIM
    h_stride = l_stride // 2
    out_blk = CHUNK * 8
    mesh = plsc.VectorSubcoreMesh(
        core_axis_name="c", subcore_axis_name="s",
        num_cores=NUM_CORES, num_subcores=NUM_SUBCORES)

    @functools.partial(
        pl.kernel,
        out_type=jax.ShapeDtypeStruct((n * EMB_DIM,), jnp.float32),
        mesh=mesh,
        compiler_params=pltpu.CompilerParams(
            needs_layout_passes=False, use_tc_tiling_on_sc=False),
        scratch_types=[
            pltpu.VMEM((24 * EMB_DIM,), jnp.float32),
            pltpu.VMEM((4 * EMB_DIM,), jnp.float32),
            pltpu.VMEM((8 * EMB_DIM,), jnp.float32),
            pltpu.VMEM((COMB_ROWS * EMB_DIM,), jnp.float32),
            pltpu.VMEM((2, CHUNK), jnp.int32),
            pltpu.VMEM((2, CHUNK), jnp.int32),
            pltpu.VMEM((2, CHUNK), jnp.int32),
            pltpu.VMEM((2, CHUNK, EMB_DIM), jnp.float32),
            pltpu.VMEM((2 * 2 * out_blk,), jnp.float32),
            pltpu.SemaphoreType.DMA((2,)),
            pltpu.SemaphoreType.DMA((2,)),
            pltpu.SemaphoreType.DMA((2,)),
            pltpu.SemaphoreType.DMA((2,)),
        ],
    )
    def sc_fn(packed_hbm, loc_hbm, hour_hbm, min_hbm,
              modetab_hbm, out_hbm, hour_v, min_v, modetab_v, comb_v,
              idx_v, time_v, mode_v, rows_v, outb_v, sem_idx, sem_tm, sem_g,
              sem_out):
        iota = lax.iota(jnp.int32, LANES)

        pltpu.sync_copy(hour_hbm, hour_v)
        pltpu.sync_copy(min_hbm, min_v)
        pltpu.sync_copy(modetab_hbm, modetab_v)

        def build_comb(i, carry):
            h = i // 32
            mn = (i // 8) % 4
            md = i % 8
            row = (plsc.load_gather(hour_v, [_splat(h * EMB_DIM) + iota])
                   + plsc.load_gather(min_v, [_splat(mn * EMB_DIM) + iota])
                   + plsc.load_gather(modetab_v, [_splat(md * EMB_DIM) + iota]))
            plsc.store_scatter(comb_v, [_splat(i * EMB_DIM) + iota], row)
            return carry

        lax.fori_loop(0, COMB_ROWS, build_comb, 0)

        wid = lax.axis_index("s") * NUM_CORES + lax.axis_index("c")
        kbase = wid * n_chunks

        def idx_copy(c, s):
            k = kbase + c
            off = (k // s_per_l) * b_dim + (k % s_per_l) * CHUNK
            return pltpu.make_async_copy(
                packed_hbm.at[pl.ds(off, CHUNK)], idx_v.at[s], sem_idx.at[s])

        def tm_copies(c, s):
            k = kbase + c
            off = (k // s_per_l) * b_dim + (k % s_per_l) * CHUNK
            return (
                pltpu.make_async_copy(
                    packed_hbm.at[pl.ds(n + off, CHUNK)], time_v.at[s],
                    sem_tm.at[s]),
                pltpu.make_async_copy(
                    packed_hbm.at[pl.ds(2 * n + off, CHUNK)], mode_v.at[s],
                    sem_tm.at[s]),
            )

        def gather_copy(s):
            return pltpu.make_async_copy(
                loc_hbm.at[idx_v.at[s]], rows_v.at[s], sem_g.at[s])

        def out_copies(c, s):
            k = kbase + c
            off = (k // s_per_l) * l_stride + (k % s_per_l) * out_blk
            return (
                pltpu.make_async_copy(
                    outb_v.at[pl.ds(s * 2 * out_blk, out_blk)],
                    out_hbm.at[pl.ds(off, out_blk)], sem_out.at[s]),
                pltpu.make_async_copy(
                    outb_v.at[pl.ds((s * 2 + 1) * out_blk, out_blk)],
                    out_hbm.at[pl.ds(off + h_stride, out_blk)], sem_out.at[s]),
            )

        idx_copy(0, 0).start()
        idx_copy(1, 1).start()
        for cp in tm_copies(0, 0):
            cp.start()
        for cp in tm_copies(1, 1):
            cp.start()
        idx_copy(0, 0).wait()
        gather_copy(0).start()

        def chunk_body(c, carry):
            buf = c % 2
            nxt = 1 - buf

            @pl.when(c + 1 < n_chunks)
            def _():
                idx_copy(c + 1, nxt).wait()
                gather_copy(nxt).start()

            gather_copy(buf).wait()

            @pl.when(c + 2 < n_chunks)
            def _():
                idx_copy(c + 2, buf).start()

            for cp in tm_copies(c, buf):
                cp.wait()

            @pl.when(c >= 2)
            def _():
                for cp in out_copies(c - 2, buf):
                    cp.wait()

            rows_2d = rows_v.at[buf]

            @plsc.parallel_loop(0, CHUNK // LANES, unroll=2)
            def group_body(g):
                tvec = time_v[buf, pl.ds(g * LANES, LANES)]
                mvec = mode_v[buf, pl.ds(g * LANES, LANES)]
                cbase = (tvec * 8 + mvec) * EMB_DIM
                rowids = g * LANES + iota
                obase = (buf * 2 * out_blk + (g // 8) * 1024
                         + (g % 8) * LANES)
                cols = []
                for e in range(EMB_DIM):
                    cols.append(
                        plsc.load_gather(rows_2d, [rowids, _splat(e)])
                        + plsc.load_gather(comb_v, [cbase + e]))
                for h in range(2):
                    for e_lo in range(8):
                        outb_v[pl.ds(obase + h * out_blk + e_lo * 128,
                                     LANES)] = cols[h * 8 + e_lo]

            for cp in out_copies(c, buf):
                cp.start()

            @pl.when(c + 2 < n_chunks)
            def _():
                for cp in tm_copies(c + 2, buf):
                    cp.start()
            return carry

        lax.fori_loop(0, n_chunks, chunk_body, 0)
        for cp in out_copies(n_chunks - 2, n_chunks % 2):
            cp.wait()
        for cp in out_copies(n_chunks - 1, 1 - n_chunks % 2):
            cp.wait()

    return sc_fn


def kernel(src, time, mode, emb_loc, emb_mode, emb_hour, emb_min):
    b, l = src.shape
    packed = jnp.concatenate([src.T.reshape(-1).astype(jnp.int32),
                              time.T.reshape(-1).astype(jnp.int32),
                              mode.T.reshape(-1).astype(jnp.int32)])
    out1d = _build_sc_call(b, l)(packed,
                                 emb_loc.astype(jnp.float32),
                                 emb_hour.astype(jnp.float32).reshape(-1),
                                 emb_min.astype(jnp.float32).reshape(-1),
                                 emb_mode.astype(jnp.float32).reshape(-1))
    x5 = out1d.reshape(l, 2, b // 128, 8, 128)
    return x5.transpose(2, 4, 0, 1, 3).reshape(b, l, EMB_DIM)


# speedup vs baseline: 1.0436x; 1.0436x over previous
import functools

import jax
import jax.numpy as jnp
from jax import lax
from jax.experimental import pallas as pl
from jax.experimental.pallas import tpu as pltpu
from jax.experimental.pallas import tpu_sc as plsc

EMB_DIM = 16
LANES = 16
NUM_CORES = 2
NUM_SUBCORES = 16
NUM_WORKERS = NUM_CORES * NUM_SUBCORES
CHUNK = 1024
COMB_ROWS = 96 * 8


def _splat(v):
    return jnp.full((LANES,), v, jnp.int32)


@functools.lru_cache(maxsize=None)
def _build_sc_call(b_dim, l_dim):
    n = b_dim * l_dim
    assert b_dim % CHUNK == 0 and EMB_DIM == 16 and b_dim % 128 == 0
    s_per_l = b_dim // CHUNK
    total_chunks = l_dim * s_per_l
    assert total_chunks % NUM_WORKERS == 0
    n_chunks = total_chunks // NUM_WORKERS
    l_stride = b_dim * EMB_DIM
    h_stride = l_stride // 2
    out_blk = CHUNK * 8
    mesh = plsc.VectorSubcoreMesh(
        core_axis_name="c", subcore_axis_name="s",
        num_cores=NUM_CORES, num_subcores=NUM_SUBCORES)

    @functools.partial(
        pl.kernel,
        out_type=jax.ShapeDtypeStruct((n * EMB_DIM,), jnp.float32),
        mesh=mesh,
        compiler_params=pltpu.CompilerParams(
            needs_layout_passes=False, use_tc_tiling_on_sc=False),
        scratch_types=[
            pltpu.VMEM((24 * EMB_DIM,), jnp.float32),
            pltpu.VMEM((4 * EMB_DIM,), jnp.float32),
            pltpu.VMEM((8 * EMB_DIM,), jnp.float32),
            pltpu.VMEM((COMB_ROWS * EMB_DIM,), jnp.float32),
            pltpu.VMEM((2, CHUNK), jnp.int32),
            pltpu.VMEM((2, CHUNK), jnp.int32),
            pltpu.VMEM((2, CHUNK), jnp.int32),
            pltpu.VMEM((2, CHUNK, EMB_DIM), jnp.float32),
            pltpu.VMEM((2 * 2 * out_blk,), jnp.float32),
            pltpu.SemaphoreType.DMA((2,)),
            pltpu.SemaphoreType.DMA((2,)),
            pltpu.SemaphoreType.DMA((2,)),
            pltpu.SemaphoreType.DMA((2,)),
        ],
    )
    def sc_fn(src_hbm, time_hbm, mode_hbm, loc_hbm, hour_hbm, min_hbm,
              modetab_hbm, out_hbm, hour_v, min_v, modetab_v, comb_v,
              idx_v, time_v, mode_v, rows_v, outb_v, sem_idx, sem_tm, sem_g,
              sem_out):
        iota = lax.iota(jnp.int32, LANES)

        pltpu.sync_copy(hour_hbm, hour_v)
        pltpu.sync_copy(min_hbm, min_v)
        pltpu.sync_copy(modetab_hbm, modetab_v)

        def build_comb(i, carry):
            h = i // 32
            mn = (i // 8) % 4
            md = i % 8
            row = (plsc.load_gather(hour_v, [_splat(h * EMB_DIM) + iota])
                   + plsc.load_gather(min_v, [_splat(mn * EMB_DIM) + iota])
                   + plsc.load_gather(modetab_v, [_splat(md * EMB_DIM) + iota]))
            plsc.store_scatter(comb_v, [_splat(i * EMB_DIM) + iota], row)
            return carry

        lax.fori_loop(0, COMB_ROWS, build_comb, 0)

        wid = lax.axis_index("s") * NUM_CORES + lax.axis_index("c")
        kbase = wid * n_chunks

        def idx_copy(c, s):
            k = kbase + c
            off = (k // s_per_l) * b_dim + (k % s_per_l) * CHUNK
            return pltpu.make_async_copy(
                src_hbm.at[pl.ds(off, CHUNK)], idx_v.at[s], sem_idx.at[s])

        def tm_copies(c, s):
            k = kbase + c
            off = (k // s_per_l) * b_dim + (k % s_per_l) * CHUNK
            return (
                pltpu.make_async_copy(
                    time_hbm.at[pl.ds(off, CHUNK)], time_v.at[s], sem_tm.at[s]),
                pltpu.make_async_copy(
                    mode_hbm.at[pl.ds(off, CHUNK)], mode_v.at[s], sem_tm.at[s]),
            )

        def gather_copy(s):
            return pltpu.make_async_copy(
                loc_hbm.at[idx_v.at[s]], rows_v.at[s], sem_g.at[s])

        def out_copies(c, s):
            k = kbase + c
            off = (k // s_per_l) * l_stride + (k % s_per_l) * out_blk
            return (
                pltpu.make_async_copy(
                    outb_v.at[pl.ds(s * 2 * out_blk, out_blk)],
                    out_hbm.at[pl.ds(off, out_blk)], sem_out.at[s]),
                pltpu.make_async_copy(
                    outb_v.at[pl.ds((s * 2 + 1) * out_blk, out_blk)],
                    out_hbm.at[pl.ds(off + h_stride, out_blk)], sem_out.at[s]),
            )

        idx_copy(0, 0).start()
        idx_copy(1, 1).start()
        for cp in tm_copies(0, 0):
            cp.start()
        for cp in tm_copies(1, 1):
            cp.start()
        idx_copy(0, 0).wait()
        gather_copy(0).start()

        def chunk_body(c, carry):
            buf = c % 2
            nxt = 1 - buf

            @pl.when(c + 1 < n_chunks)
            def _():
                idx_copy(c + 1, nxt).wait()
                gather_copy(nxt).start()

            gather_copy(buf).wait()

            @pl.when(c + 2 < n_chunks)
            def _():
                idx_copy(c + 2, buf).start()

            for cp in tm_copies(c, buf):
                cp.wait()

            @pl.when(c >= 2)
            def _():
                for cp in out_copies(c - 2, buf):
                    cp.wait()

            rows_2d = rows_v.at[buf]

            @plsc.parallel_loop(0, CHUNK // LANES, unroll=2)
            def group_body(g):
                tvec = time_v[buf, pl.ds(g * LANES, LANES)]
                mvec = mode_v[buf, pl.ds(g * LANES, LANES)]
                cbase = (tvec * 8 + mvec) * EMB_DIM
                rowids = g * LANES + iota
                obase = (buf * 2 * out_blk + (g // 8) * 1024
                         + (g % 8) * LANES)
                cols = []
                for e in range(EMB_DIM):
                    cols.append(
                        plsc.load_gather(rows_2d, [rowids, _splat(e)])
                        + plsc.load_gather(comb_v, [cbase + e]))
                for h in range(2):
                    for e_lo in range(8):
                        outb_v[pl.ds(obase + h * out_blk + e_lo * 128,
                                     LANES)] = cols[h * 8 + e_lo]

            for cp in out_copies(c, buf):
                cp.start()

            @pl.when(c + 2 < n_chunks)
            def _():
                for cp in tm_copies(c + 2, buf):
                    cp.start()
            return carry

        lax.fori_loop(0, n_chunks, chunk_body, 0)
        for cp in out_copies(n_chunks - 2, n_chunks % 2):
            cp.wait()
        for cp in out_copies(n_chunks - 1, 1 - n_chunks % 2):
            cp.wait()

    return sc_fn


def kernel(src, time, mode, emb_loc, emb_mode, emb_hour, emb_min):
    b, l = src.shape
    src_f = src.T.reshape(-1).astype(jnp.int32)
    time_f = time.T.reshape(-1).astype(jnp.int32)
    mode_f = mode.T.reshape(-1).astype(jnp.int32)
    out1d = _build_sc_call(b, l)(src_f, time_f, mode_f,
                                 emb_loc.astype(jnp.float32),
                                 emb_hour.astype(jnp.float32).reshape(-1),
                                 emb_min.astype(jnp.float32).reshape(-1),
                                 emb_mode.astype(jnp.float32).reshape(-1))
    x5 = out1d.reshape(l, 2, b // 128, 8, 128)
    return x5.transpose(2, 4, 0, 1, 3).reshape(b, l, EMB_DIM)


# speedup vs baseline: 1.0510x; 1.0070x over previous
import functools

import jax
import jax.numpy as jnp
from jax import lax
from jax.experimental import pallas as pl
from jax.experimental.pallas import tpu as pltpu
from jax.experimental.pallas import tpu_sc as plsc

EMB_DIM = 16
LANES = 16
NUM_CORES = 2
NUM_SUBCORES = 16
NUM_WORKERS = NUM_CORES * NUM_SUBCORES
CHUNK = 1024
COMB_ROWS = 96 * 8


def _splat(v):
    return jnp.full((LANES,), v, jnp.int32)


@functools.lru_cache(maxsize=None)
def _build_sc_call(b_dim, l_dim):
    n = b_dim * l_dim
    assert b_dim % CHUNK == 0 and EMB_DIM == 16 and b_dim % 128 == 0
    s_per_l = b_dim // CHUNK
    total_chunks = l_dim * s_per_l
    assert total_chunks % NUM_WORKERS == 0
    n_chunks = total_chunks // NUM_WORKERS
    l_stride = b_dim * EMB_DIM
    h_stride = l_stride // 2
    out_blk = CHUNK * 8
    mesh = plsc.VectorSubcoreMesh(
        core_axis_name="c", subcore_axis_name="s",
        num_cores=NUM_CORES, num_subcores=NUM_SUBCORES)

    @functools.partial(
        pl.kernel,
        out_type=jax.ShapeDtypeStruct((n * EMB_DIM,), jnp.float32),
        mesh=mesh,
        compiler_params=pltpu.CompilerParams(
            needs_layout_passes=False, use_tc_tiling_on_sc=False),
        scratch_types=[
            pltpu.VMEM((24 * EMB_DIM,), jnp.float32),
            pltpu.VMEM((4 * EMB_DIM,), jnp.float32),
            pltpu.VMEM((8 * EMB_DIM,), jnp.float32),
            pltpu.VMEM((COMB_ROWS * EMB_DIM,), jnp.float32),
            pltpu.VMEM((2, CHUNK), jnp.int32),
            pltpu.VMEM((2, CHUNK), jnp.int32),
            pltpu.VMEM((2, CHUNK), jnp.int32),
            pltpu.VMEM((2, CHUNK, EMB_DIM), jnp.float32),
            pltpu.VMEM((2 * 2 * out_blk,), jnp.float32),
            pltpu.SemaphoreType.DMA((2,)),
            pltpu.SemaphoreType.DMA((2,)),
            pltpu.SemaphoreType.DMA((2,)),
            pltpu.SemaphoreType.DMA((2,)),
        ],
    )
    def sc_fn(src_hbm, time_hbm, mode_hbm, loc_hbm, hour_hbm, min_hbm,
              modetab_hbm, out_hbm, hour_v, min_v, modetab_v, comb_v,
              idx_v, time_v, mode_v, rows_v, outb_v, sem_idx, sem_tm, sem_g,
              sem_out):
        iota = lax.iota(jnp.int32, LANES)

        pltpu.sync_copy(hour_hbm, hour_v)
        pltpu.sync_copy(min_hbm, min_v)
        pltpu.sync_copy(modetab_hbm, modetab_v)

        def build_comb(i, carry):
            h = i // 32
            mn = (i // 8) % 4
            md = i % 8
            row = (plsc.load_gather(hour_v, [_splat(h * EMB_DIM) + iota])
                   + plsc.load_gather(min_v, [_splat(mn * EMB_DIM) + iota])
                   + plsc.load_gather(modetab_v, [_splat(md * EMB_DIM) + iota]))
            plsc.store_scatter(comb_v, [_splat(i * EMB_DIM) + iota], row)
            return carry

        lax.fori_loop(0, COMB_ROWS, build_comb, 0)

        wid = lax.axis_index("s") * NUM_CORES + lax.axis_index("c")
        kbase = wid * n_chunks

        def _seg_base(c):
            k = kbase + c
            l = k // s_per_l
            t0 = (l // 8) * (b_dim // 128) + (k % s_per_l) * 8
            return t0 * 1024 + (l % 8) * 128

        def idx_copy(c, s):
            base = _seg_base(c)
            return [pltpu.make_async_copy(
                src_hbm.at[pl.ds(base + j * 1024, 128)],
                idx_v.at[s, pl.ds(j * 128, 128)], sem_idx.at[s])
                for j in range(8)]

        def tm_copies(c, s):
            base = _seg_base(c)
            cps = []
            for j in range(8):
                cps.append(pltpu.make_async_copy(
                    time_hbm.at[pl.ds(base + j * 1024, 128)],
                    time_v.at[s, pl.ds(j * 128, 128)], sem_tm.at[s]))
                cps.append(pltpu.make_async_copy(
                    mode_hbm.at[pl.ds(base + j * 1024, 128)],
                    mode_v.at[s, pl.ds(j * 128, 128)], sem_tm.at[s]))
            return cps

        def gather_copy(s):
            return pltpu.make_async_copy(
                loc_hbm.at[idx_v.at[s]], rows_v.at[s], sem_g.at[s])

        def out_copies(c, s):
            k = kbase + c
            off = (k // s_per_l) * l_stride + (k % s_per_l) * out_blk
            return (
                pltpu.make_async_copy(
                    outb_v.at[pl.ds(s * 2 * out_blk, out_blk)],
                    out_hbm.at[pl.ds(off, out_blk)], sem_out.at[s]),
                pltpu.make_async_copy(
                    outb_v.at[pl.ds((s * 2 + 1) * out_blk, out_blk)],
                    out_hbm.at[pl.ds(off + h_stride, out_blk)], sem_out.at[s]),
            )

        for cp in idx_copy(0, 0):
            cp.start()
        for cp in idx_copy(1, 1):
            cp.start()
        for cp in tm_copies(0, 0):
            cp.start()
        for cp in tm_copies(1, 1):
            cp.start()
        for cp in idx_copy(0, 0):
            cp.wait()
        gather_copy(0).start()

        def chunk_body(c, carry):
            buf = c % 2
            nxt = 1 - buf

            @pl.when(c + 1 < n_chunks)
            def _():
                for cp in idx_copy(c + 1, nxt):
                    cp.wait()
                gather_copy(nxt).start()

            gather_copy(buf).wait()

            @pl.when(c + 2 < n_chunks)
            def _():
                for cp in idx_copy(c + 2, buf):
                    cp.start()

            for cp in tm_copies(c, buf):
                cp.wait()

            @pl.when(c >= 2)
            def _():
                for cp in out_copies(c - 2, buf):
                    cp.wait()

            rows_2d = rows_v.at[buf]

            @plsc.parallel_loop(0, CHUNK // LANES, unroll=2)
            def group_body(g):
                tvec = time_v[buf, pl.ds(g * LANES, LANES)]
                mvec = mode_v[buf, pl.ds(g * LANES, LANES)]
                cbase = (tvec * 8 + mvec) * EMB_DIM
                rowids = g * LANES + iota
                obase = (buf * 2 * out_blk + (g // 8) * 1024
                         + (g % 8) * LANES)
                cols = []
                for e in range(EMB_DIM):
                    cols.append(
                        plsc.load_gather(rows_2d, [rowids, _splat(e)])
                        + plsc.load_gather(comb_v, [cbase + e]))
                for h in range(2):
                    for e_lo in range(8):
                        outb_v[pl.ds(obase + h * out_blk + e_lo * 128,
                                     LANES)] = cols[h * 8 + e_lo]

            for cp in out_copies(c, buf):
                cp.start()

            @pl.when(c + 2 < n_chunks)
            def _():
                for cp in tm_copies(c + 2, buf):
                    cp.start()
            return carry

        lax.fori_loop(0, n_chunks, chunk_body, 0)
        for cp in out_copies(n_chunks - 2, n_chunks % 2):
            cp.wait()
        for cp in out_copies(n_chunks - 1, 1 - n_chunks % 2):
            cp.wait()

    return sc_fn


def kernel(src, time, mode, emb_loc, emb_mode, emb_hour, emb_min):
    b, l = src.shape

    def _flat_physical(x):
        x4 = x.astype(jnp.int32).reshape(b // 128, 128, l // 8, 8)
        return x4.transpose(2, 0, 3, 1).reshape(-1)

    src_f = _flat_physical(src)
    time_f = _flat_physical(time)
    mode_f = _flat_physical(mode)
    out1d = _build_sc_call(b, l)(src_f, time_f, mode_f,
                                 emb_loc.astype(jnp.float32),
                                 emb_hour.astype(jnp.float32).reshape(-1),
                                 emb_min.astype(jnp.float32).reshape(-1),
                                 emb_mode.astype(jnp.float32).reshape(-1))
    x5 = out1d.reshape(l, 2, b // 128, 8, 128)
    return x5.transpose(2, 4, 0, 1, 3).reshape(b, l, EMB_DIM)
